# baseline (speedup 1.0000x reference)
import functools

import jax
import jax.numpy as jnp
from jax.experimental import pallas as pl

G, T, H, E, C = 2, 2048, 2048, 16, 128
TBLK_A = 1024
TBLK_C = 512


def _probs_body(x_ref, w_ref, b_ref, probs_ref, z_ref):
    g = pl.program_id(0)
    tb = pl.program_id(1)
    x = x_ref[0]
    w = w_ref[...]
    b = b_ref[...]
    logits = jax.lax.dot_general(
        w, x, dimension_numbers=(((0,), (1,)), ((), ())),
        preferred_element_type=jnp.float32)
    logits = logits + b.reshape(E, 1)
    m = jnp.max(logits, axis=0, keepdims=True)
    ex = jnp.exp(logits - m)
    s = jnp.sum(ex, axis=0, keepdims=True)
    probs_ref[0] = ex / s
    lse = m + jnp.log(s)
    zpart = jnp.sum(lse * lse).reshape(1, 1)

    @pl.when(jnp.logical_and(g == 0, tb == 0))
    def _():
        z_ref[...] = jnp.zeros_like(z_ref)

    z_ref[...] += zpart


def _first(av, ai, bv, bi):
    return (av > bv) | ((av == bv) & (ai < bi))


def _cex(v, i, islow, j, keepmask):
    pv = jnp.where(islow, jnp.roll(v, -j, 1), jnp.roll(v, j, 1))
    pi = jnp.where(islow, jnp.roll(i, -j, 1), jnp.roll(i, j, 1))
    sf = _first(v, i, pv, pi)
    keep = sf == keepmask
    return jnp.where(keep, v, pv), jnp.where(keep, i, pi)


def _topk_body(p_ref, ei_ref, eg_ref):
    rows = G * E
    v = p_ref[...]
    lane = jax.lax.broadcasted_iota(jnp.int32, (rows, T), 1)
    i = lane
    want = lane < (T // 2)
    islow_by_j = {j: (lane & j) == 0 for j in (1, 2, 4, 8, 16, 32, 64)}
    for k in (2, 4, 8, 16, 32, 64, 128):
        desc = want if k == 128 else want ^ ((lane & k) != 0)
        j = k // 2
        while j >= 1:
            islow = islow_by_j[j]
            v, i = _cex(v, i, islow, j, islow == desc)
            j //= 2
    w = T
    while w > C:
        h = w // 2
        f = _first(v[:, :h], i[:, :h], v[:, h:w], i[:, h:w])
        v = jnp.where(f, v[:, :h], v[:, h:w])
        i = jnp.where(f, i[:, :h], i[:, h:w])
        desc_h = lane[:, :h] < max(h // 2, C)
        for j in (64, 32, 16, 8, 4, 2, 1):
            islow = islow_by_j[j][:, :h]
            v, i = _cex(v, i, islow, j, islow == desc_h)
        w = h
    ei_ref[...] = i
    eg_ref[...] = v


def _mask_body(ei_ref, eg_ref, disp_ref, comb_ref):
    tb = pl.program_id(1)
    t0 = tb * TBLK_C
    ti = jax.lax.broadcasted_iota(jnp.int32, (TBLK_C, E, C), 0) + t0
    hit = ei_ref[0][None, :, :] == ti
    disp_ref[0] = jnp.where(hit, 1.0, 0.0).astype(jnp.float32)
    comb_ref[0] = jnp.where(hit, eg_ref[0][None, :, :], 0.0).astype(jnp.float32)


@functools.partial(jax.jit, static_argnums=())
def _run(x, w, b):
    probs_t, zsum = pl.pallas_call(
        _probs_body,
        grid=(G, T // TBLK_A),
        in_specs=[
            pl.BlockSpec((1, TBLK_A, H), lambda g, tb: (g, tb, 0)),
            pl.BlockSpec((H, E), lambda g, tb: (0, 0)),
            pl.BlockSpec((1, E), lambda g, tb: (0, 0)),
        ],
        out_specs=[
            pl.BlockSpec((1, E, TBLK_A), lambda g, tb: (g, 0, tb)),
            pl.BlockSpec((1, 1), lambda g, tb: (0, 0)),
        ],
        out_shape=[
            jax.ShapeDtypeStruct((G, E, T), jnp.float32),
            jax.ShapeDtypeStruct((1, 1), jnp.float32),
        ],
    )(x, w, b.reshape(1, E))

    ei, eg = pl.pallas_call(
        _topk_body,
        in_specs=[pl.BlockSpec((G * E, T), lambda: (0, 0))],
        out_specs=[
            pl.BlockSpec((G * E, C), lambda: (0, 0)),
            pl.BlockSpec((G * E, C), lambda: (0, 0)),
        ],
        out_shape=[
            jax.ShapeDtypeStruct((G * E, C), jnp.int32),
            jax.ShapeDtypeStruct((G * E, C), jnp.float32),
        ],
    )(probs_t.reshape(G * E, T))

    disp, comb = pl.pallas_call(
        _mask_body,
        grid=(G, T // TBLK_C),
        in_specs=[
            pl.BlockSpec((1, E, C), lambda g, tb: (g, 0, 0)),
            pl.BlockSpec((1, E, C), lambda g, tb: (g, 0, 0)),
        ],
        out_specs=[
            pl.BlockSpec((1, TBLK_C, E, C), lambda g, tb: (g, tb, 0, 0)),
            pl.BlockSpec((1, TBLK_C, E, C), lambda g, tb: (g, tb, 0, 0)),
        ],
        out_shape=[
            jax.ShapeDtypeStruct((G, T, E, C), jnp.float32),
            jax.ShapeDtypeStruct((G, T, E, C), jnp.float32),
        ],
    )(ei.reshape(G, E, C), eg.reshape(G, E, C))

    z_loss = zsum[0, 0] / (G * T)
    return disp, comb, z_loss


def kernel(inputs, kernel, bias, expert_capacity):
    del expert_capacity
    return _run(inputs, kernel, bias)


# speedup vs baseline: 1.0281x; 1.0281x over previous
import functools

import jax
import jax.numpy as jnp
from jax.experimental import pallas as pl

G, T, H, E, C = 2, 2048, 2048, 16, 128
TBLK_A = 1024
TBLK_C = 512


def _probs_body(x_ref, w_ref, b_ref, probs_ref, z_ref):
    g = pl.program_id(0)
    tb = pl.program_id(1)
    x = x_ref[0]
    w = w_ref[...]
    b = b_ref[...]
    logits = jax.lax.dot_general(
        w, x, dimension_numbers=(((0,), (1,)), ((), ())),
        preferred_element_type=jnp.float32)
    logits = logits + b.reshape(E, 1)
    m = jnp.max(logits, axis=0, keepdims=True)
    ex = jnp.exp(logits - m)
    s = jnp.sum(ex, axis=0, keepdims=True)
    probs_ref[0] = ex / s
    lse = m + jnp.log(s)
    zpart = jnp.sum(lse * lse).reshape(1, 1)

    @pl.when(jnp.logical_and(g == 0, tb == 0))
    def _():
        z_ref[...] = jnp.zeros_like(z_ref)

    z_ref[...] += zpart


def _first(av, ai, bv, bi):
    return (av > bv) | ((av == bv) & (ai < bi))


def _cex(v, i, islow, j, keepmask):
    pv = jnp.where(islow, jnp.roll(v, -j, 1), jnp.roll(v, j, 1))
    pi = jnp.where(islow, jnp.roll(i, -j, 1), jnp.roll(i, j, 1))
    sf = _first(v, i, pv, pi)
    keep = sf == keepmask
    return jnp.where(keep, v, pv), jnp.where(keep, i, pi)


def _topk_mask_body(p_ref, ei_ref, eg_ref, disp_ref, comb_ref):
    s = pl.program_id(0)

    @pl.when(s == 0)
    def _():
        _topk_core(p_ref, ei_ref, eg_ref)

    @pl.when(s > 0)
    def _():
        nblk = T // TBLK_C
        g = (s - 1) // nblk
        tb = (s - 1) % nblk
        t0 = tb * TBLK_C
        ti = jax.lax.broadcasted_iota(jnp.int32, (TBLK_C, E, C), 0) + t0
        for gs in range(G):
            @pl.when(g == gs)
            def _():
                ei = ei_ref[gs * E:(gs + 1) * E, :]
                eg = eg_ref[gs * E:(gs + 1) * E, :]
                hit = ei[None, :, :] == ti
                disp_ref[0] = jnp.where(hit, 1.0, 0.0).astype(jnp.float32)
                comb_ref[0] = jnp.where(
                    hit, eg[None, :, :], 0.0).astype(jnp.float32)


def _topk_core(p_ref, ei_ref, eg_ref):
    rows = G * E
    v = p_ref[...]
    lane = jax.lax.broadcasted_iota(jnp.int32, (rows, T), 1)
    i = lane
    want = lane < (T // 2)
    islow_by_j = {j: (lane & j) == 0 for j in (1, 2, 4, 8, 16, 32, 64)}
    for k in (2, 4, 8, 16, 32, 64, 128):
        desc = want if k == 128 else want ^ ((lane & k) != 0)
        j = k // 2
        while j >= 1:
            islow = islow_by_j[j]
            v, i = _cex(v, i, islow, j, islow == desc)
            j //= 2
    w = T
    while w > C:
        h = w // 2
        f = _first(v[:, :h], i[:, :h], v[:, h:w], i[:, h:w])
        v = jnp.where(f, v[:, :h], v[:, h:w])
        i = jnp.where(f, i[:, :h], i[:, h:w])
        desc_h = lane[:, :h] < max(h // 2, C)
        for j in (64, 32, 16, 8, 4, 2, 1):
            islow = islow_by_j[j][:, :h]
            v, i = _cex(v, i, islow, j, islow == desc_h)
        w = h
    ei_ref[...] = i
    eg_ref[...] = v


def _mask_body(ei_ref, eg_ref, disp_ref, comb_ref):
    tb = pl.program_id(1)
    t0 = tb * TBLK_C
    ti = jax.lax.broadcasted_iota(jnp.int32, (TBLK_C, E, C), 0) + t0
    hit = ei_ref[0][None, :, :] == ti
    disp_ref[0] = jnp.where(hit, 1.0, 0.0).astype(jnp.float32)
    comb_ref[0] = jnp.where(hit, eg_ref[0][None, :, :], 0.0).astype(jnp.float32)


@functools.partial(jax.jit, static_argnums=())
def _run(x, w, b):
    probs_t, zsum = pl.pallas_call(
        _probs_body,
        grid=(G, T // TBLK_A),
        in_specs=[
            pl.BlockSpec((1, TBLK_A, H), lambda g, tb: (g, tb, 0)),
            pl.BlockSpec((H, E), lambda g, tb: (0, 0)),
            pl.BlockSpec((1, E), lambda g, tb: (0, 0)),
        ],
        out_specs=[
            pl.BlockSpec((1, E, TBLK_A), lambda g, tb: (g, 0, tb)),
            pl.BlockSpec((1, 1), lambda g, tb: (0, 0)),
        ],
        out_shape=[
            jax.ShapeDtypeStruct((G, E, T), jnp.float32),
            jax.ShapeDtypeStruct((1, 1), jnp.float32),
        ],
    )(x, w, b.reshape(1, E))

    nblk = T // TBLK_C

    def _mask_idx(s):
        sm = jnp.maximum(s - 1, 0)
        return (sm // nblk, sm % nblk, 0, 0)

    _, _, disp, comb = pl.pallas_call(
        _topk_mask_body,
        grid=(1 + G * nblk,),
        in_specs=[pl.BlockSpec((G * E, T), lambda s: (0, 0))],
        out_specs=[
            pl.BlockSpec((G * E, C), lambda s: (0, 0)),
            pl.BlockSpec((G * E, C), lambda s: (0, 0)),
            pl.BlockSpec((1, TBLK_C, E, C), _mask_idx),
            pl.BlockSpec((1, TBLK_C, E, C), _mask_idx),
        ],
        out_shape=[
            jax.ShapeDtypeStruct((G * E, C), jnp.int32),
            jax.ShapeDtypeStruct((G * E, C), jnp.float32),
            jax.ShapeDtypeStruct((G, T, E, C), jnp.float32),
            jax.ShapeDtypeStruct((G, T, E, C), jnp.float32),
        ],
    )(probs_t.reshape(G * E, T))

    z_loss = zsum[0, 0] / (G * T)
    return disp, comb, z_loss


def kernel(inputs, kernel, bias, expert_capacity):
    del expert_capacity
    return _run(inputs, kernel, bias)


# speedup vs baseline: 1.0308x; 1.0026x over previous
import functools

import jax
import jax.numpy as jnp
from jax.experimental import pallas as pl

G, T, H, E, C = 2, 2048, 2048, 16, 128
TBLK_A = 1024
TBLK_C = 512


def _probs_body(x_ref, w_ref, b_ref, probs_ref, z_ref):
    g = pl.program_id(0)
    tb = pl.program_id(1)
    x = x_ref[0]
    w = w_ref[...]
    b = b_ref[...]
    logits = jax.lax.dot_general(
        w, x, dimension_numbers=(((0,), (1,)), ((), ())),
        preferred_element_type=jnp.float32)
    logits = logits + b.reshape(E, 1)
    m = jnp.max(logits, axis=0, keepdims=True)
    ex = jnp.exp(logits - m)
    s = jnp.sum(ex, axis=0, keepdims=True)
    probs_ref[0] = ex / s
    lse = m + jnp.log(s)
    zpart = jnp.sum(lse * lse).reshape(1, 1)

    @pl.when(jnp.logical_and(g == 0, tb == 0))
    def _():
        z_ref[...] = jnp.zeros_like(z_ref)

    z_ref[...] += zpart


def _first(av, ai, bv, bi):
    return (av > bv) | ((av == bv) & (ai < bi))


def _cex(v, i, islow, j, keepmask):
    pv = jnp.where(islow, jnp.roll(v, -j, 1), jnp.roll(v, j, 1))
    pi = jnp.where(islow, jnp.roll(i, -j, 1), jnp.roll(i, j, 1))
    sf = _first(v, i, pv, pi)
    keep = sf == keepmask
    return jnp.where(keep, v, pv), jnp.where(keep, i, pi)


def _topk_mask_body(p_ref, ei_ref, eg_ref, disp_ref, comb_ref):
    s = pl.program_id(0)

    @pl.when(s == 0)
    def _():
        _topk_core(p_ref, ei_ref, eg_ref)

    @pl.when(s > 0)
    def _():
        nblk = T // TBLK_C
        g = (s - 1) // nblk
        tb = (s - 1) % nblk
        t0 = tb * TBLK_C
        ti = jax.lax.broadcasted_iota(jnp.int32, (TBLK_C, E, C), 0) + t0
        for gs in range(G):
            @pl.when(g == gs)
            def _():
                ei = ei_ref[gs * E:(gs + 1) * E, :]
                eg = eg_ref[gs * E:(gs + 1) * E, :]
                hit = ei[None, :, :] == ti
                disp_ref[0] = jnp.where(hit, 1.0, 0.0).astype(jnp.float32)
                comb_ref[0] = jnp.where(
                    hit, eg[None, :, :], 0.0).astype(jnp.float32)


def _topk_core(p_ref, ei_ref, eg_ref):
    rows = G * E
    v = p_ref[...]
    lane = jax.lax.broadcasted_iota(jnp.int32, (rows, T), 1)
    i = lane
    want = lane < (T // 2)
    islow_by_j = {j: (lane & j) == 0 for j in (1, 2, 4, 8, 16, 32, 64)}
    for k in (2, 4, 8, 16, 32, 64, 128):
        desc = want if k == 128 else want ^ ((lane & k) != 0)
        j = k // 2
        while j >= 1:
            islow = islow_by_j[j]
            v, i = _cex(v, i, islow, j, islow == desc)
            j //= 2
    w = T
    while w > C:
        h = w // 2
        f = _first(v[:, :h], i[:, :h], v[:, h:w], i[:, h:w])
        v = jnp.where(f, v[:, :h], v[:, h:w])
        i = jnp.where(f, i[:, :h], i[:, h:w])
        desc_h = lane[:, :h] < max(h // 2, C)
        for j in (64, 32, 16, 8, 4, 2, 1):
            islow = islow_by_j[j][:, :h]
            v, i = _cex(v, i, islow, j, islow == desc_h)
        w = h
    ei_ref[...] = i
    eg_ref[...] = v


@functools.partial(jax.jit, static_argnums=())
def _run(x, w, b):
    probs_t, zsum = pl.pallas_call(
        _probs_body,
        grid=(G, T // TBLK_A),
        in_specs=[
            pl.BlockSpec((1, TBLK_A, H), lambda g, tb: (g, tb, 0)),
            pl.BlockSpec((H, E), lambda g, tb: (0, 0)),
            pl.BlockSpec((1, E), lambda g, tb: (0, 0)),
        ],
        out_specs=[
            pl.BlockSpec((1, E, TBLK_A), lambda g, tb: (g, 0, tb)),
            pl.BlockSpec((1, 1), lambda g, tb: (0, 0)),
        ],
        out_shape=[
            jax.ShapeDtypeStruct((G, E, T), jnp.float32),
            jax.ShapeDtypeStruct((1, 1), jnp.float32),
        ],
    )(x, w, b.reshape(1, E))

    nblk = T // TBLK_C

    def _mask_idx(s):
        sm = jnp.maximum(s - 1, 0)
        return (sm // nblk, sm % nblk, 0, 0)

    _, _, disp, comb = pl.pallas_call(
        _topk_mask_body,
        grid=(1 + G * nblk,),
        in_specs=[pl.BlockSpec((G * E, T), lambda s: (0, 0))],
        out_specs=[
            pl.BlockSpec((G * E, C), lambda s: (0, 0)),
            pl.BlockSpec((G * E, C), lambda s: (0, 0)),
            pl.BlockSpec((1, TBLK_C, E, C), _mask_idx),
            pl.BlockSpec((1, TBLK_C, E, C), _mask_idx),
        ],
        out_shape=[
            jax.ShapeDtypeStruct((G * E, C), jnp.int32),
            jax.ShapeDtypeStruct((G * E, C), jnp.float32),
            jax.ShapeDtypeStruct((G, T, E, C), jnp.float32),
            jax.ShapeDtypeStruct((G, T, E, C), jnp.float32),
        ],
    )(probs_t.reshape(G * E, T))

    z_loss = zsum[0, 0] / (G * T)
    return disp, comb, z_loss


def kernel(inputs, kernel, bias, expert_capacity):
    del expert_capacity
    return _run(inputs, kernel, bias)


# speedup vs baseline: 1.0556x; 1.0241x over previous
import functools

import jax
import jax.numpy as jnp
from jax.experimental import pallas as pl
from jax.experimental.pallas import tpu as pltpu

G, T, H, E, C = 2, 2048, 2048, 16, 128
TBLK_A = 1024
TBLK_C = 512


def _first(av, ai, bv, bi):
    return (av > bv) | ((av == bv) & (ai < bi))


def _cex(v, i, islow, j, keepmask):
    pv = jnp.where(islow, jnp.roll(v, -j, 1), jnp.roll(v, j, 1))
    pi = jnp.where(islow, jnp.roll(i, -j, 1), jnp.roll(i, j, 1))
    sf = _first(v, i, pv, pi)
    keep = sf == keepmask
    return jnp.where(keep, v, pv), jnp.where(keep, i, pi)


def _topk_core(v, ei_ref, eg_ref):
    rows = G * E
    lane = jax.lax.broadcasted_iota(jnp.int32, (rows, T), 1)
    i = lane
    want = lane < (T // 2)
    islow_by_j = {j: (lane & j) == 0 for j in (1, 2, 4, 8, 16, 32, 64)}
    for k in (2, 4, 8, 16, 32, 64, 128):
        desc = want if k == 128 else want ^ ((lane & k) != 0)
        j = k // 2
        while j >= 1:
            islow = islow_by_j[j]
            v, i = _cex(v, i, islow, j, islow == desc)
            j //= 2
    w = T
    while w > C:
        h = w // 2
        f = _first(v[:, :h], i[:, :h], v[:, h:w], i[:, h:w])
        v = jnp.where(f, v[:, :h], v[:, h:w])
        i = jnp.where(f, i[:, :h], i[:, h:w])
        desc_h = lane[:, :h] < max(h // 2, C)
        for j in (64, 32, 16, 8, 4, 2, 1):
            islow = islow_by_j[j][:, :h]
            v, i = _cex(v, i, islow, j, islow == desc_h)
        w = h
    ei_ref[...] = i
    eg_ref[...] = v


NA = T // TBLK_A
NBLK = T // TBLK_C
S_TOPK = G * NA


def _mega_body(x_ref, w_ref, b_ref, ei_ref, eg_ref, z_ref,
               disp_ref, comb_ref, probs_s):
    s = pl.program_id(0)

    @pl.when(s < S_TOPK)
    def _():
        x = x_ref[0]
        w = w_ref[...]
        b = b_ref[...]
        logits = jax.lax.dot_general(
            w, x, dimension_numbers=(((0,), (1,)), ((), ())),
            preferred_element_type=jnp.float32)
        logits = logits + b.reshape(E, 1)
        m = jnp.max(logits, axis=0, keepdims=True)
        ex = jnp.exp(logits - m)
        sm = jnp.sum(ex, axis=0, keepdims=True)
        probs_s[pl.ds(s, 1)] = (ex / sm)[None]
        lse = m + jnp.log(sm)
        zpart = jnp.sum(lse * lse).reshape(1, 1)

        @pl.when(s == 0)
        def _():
            z_ref[...] = jnp.zeros_like(z_ref)

        z_ref[...] += zpart

    @pl.when(s == S_TOPK)
    def _():
        groups = []
        for g in range(G):
            groups.append(jnp.concatenate(
                [probs_s[g * NA + t] for t in range(NA)], axis=1))
        _topk_core(jnp.concatenate(groups, axis=0), ei_ref, eg_ref)

    @pl.when(s > S_TOPK)
    def _():
        m = s - S_TOPK - 1
        g = m // NBLK
        tb = m % NBLK
        t0 = tb * TBLK_C
        ti = jax.lax.broadcasted_iota(jnp.int32, (TBLK_C, E, C), 0) + t0
        for gs in range(G):
            @pl.when(g == gs)
            def _():
                ei = ei_ref[gs * E:(gs + 1) * E, :]
                eg = eg_ref[gs * E:(gs + 1) * E, :]
                hit = ei[None, :, :] == ti
                disp_ref[0] = jnp.where(hit, 1.0, 0.0).astype(jnp.float32)
                comb_ref[0] = jnp.where(
                    hit, eg[None, :, :], 0.0).astype(jnp.float32)


@functools.partial(jax.jit, static_argnums=())
def _run(x, w, b):
    def _mask_idx(s):
        sm = jnp.maximum(s - S_TOPK - 1, 0)
        return (sm // NBLK, sm % NBLK, 0, 0)

    def _x_idx(s):
        sc = jnp.minimum(s, S_TOPK - 1)
        return (sc // NA, sc % NA, 0)

    ei, eg, zsum, disp, comb = pl.pallas_call(
        _mega_body,
        grid=(S_TOPK + 1 + G * NBLK,),
        in_specs=[
            pl.BlockSpec((1, TBLK_A, H), _x_idx),
            pl.BlockSpec((H, E), lambda s: (0, 0)),
            pl.BlockSpec((1, E), lambda s: (0, 0)),
        ],
        out_specs=[
            pl.BlockSpec((G * E, C), lambda s: (0, 0)),
            pl.BlockSpec((G * E, C), lambda s: (0, 0)),
            pl.BlockSpec((1, 1), lambda s: (0, 0)),
            pl.BlockSpec((1, TBLK_C, E, C), _mask_idx),
            pl.BlockSpec((1, TBLK_C, E, C), _mask_idx),
        ],
        out_shape=[
            jax.ShapeDtypeStruct((G * E, C), jnp.int32),
            jax.ShapeDtypeStruct((G * E, C), jnp.float32),
            jax.ShapeDtypeStruct((1, 1), jnp.float32),
            jax.ShapeDtypeStruct((G, T, E, C), jnp.float32),
            jax.ShapeDtypeStruct((G, T, E, C), jnp.float32),
        ],
        scratch_shapes=[pltpu.VMEM((G * NA, E, TBLK_A), jnp.float32)],
    )(x, w, b.reshape(1, E))

    z_loss = zsum[0, 0] / (G * T)
    return disp, comb, z_loss


def kernel(inputs, kernel, bias, expert_capacity):
    del expert_capacity
    return _run(inputs, kernel, bias)
